# baseline (speedup 1.0000x reference)
import functools

import jax
import jax.numpy as jnp
from jax import lax
from jax.experimental import pallas as pl
from jax.experimental.pallas import tpu as pltpu
from jax.experimental.pallas import tpu_sc as plsc

B = 16384
D = 64
NC, NS, L = 2, 16, 16
NW = NC * NS
BPW = B // NW
CG = 128
NCHUNK = BPW // CG
GPC = CG // L

_mesh = plsc.VectorSubcoreMesh(core_axis_name="c", subcore_axis_name="s")


@functools.partial(
    pl.kernel,
    out_type=jax.ShapeDtypeStruct((B,), jnp.float32),
    mesh=_mesh,
    scratch_types=[
        pltpu.VMEM((BPW,), jnp.int32),
        pltpu.VMEM((BPW,), jnp.int32),
        pltpu.VMEM((BPW,), jnp.int32),
        pltpu.VMEM((BPW,), jnp.int32),
        pltpu.VMEM((2, CG, 128), jnp.float32),
        pltpu.VMEM((2, CG, 128), jnp.float32),
        pltpu.VMEM((BPW,), jnp.float32),
        pltpu.SemaphoreType.DMA((2,)),
    ],
    compiler_params=pltpu.CompilerParams(needs_layout_passes=False),
)
def _sc_dots(emb2_hbm, x1_hbm, x2_hbm, out_hbm, ix1, ix2, g1, g2,
             rows1, rows2, dots, sems):
    wid = lax.axis_index("s") * NC + lax.axis_index("c")
    base = wid * BPW

    pltpu.sync_copy(x1_hbm.at[pl.ds(base, BPW)], ix1)
    pltpu.sync_copy(x2_hbm.at[pl.ds(base, BPW)], ix2)
    half = jnp.full((L,), 500000, jnp.int32)
    hoff = jnp.full((L,), 499712, jnp.int32)
    for k in range(BPW // L):
        sl = pl.ds(k * L, L)
        v1 = ix1[sl]
        v2 = ix2[sl]
        g1[sl] = jnp.where(v1 >= half, v1 - hoff, v1)
        g2[sl] = jnp.where(v2 >= half, v2 - hoff, v2)

    def fire(j, slot):
        isl = pl.ds(j * CG, CG)
        pltpu.async_copy(emb2_hbm.at[g1.at[isl]], rows1.at[slot],
                         sems.at[slot])
        pltpu.async_copy(emb2_hbm.at[g2.at[isl]], rows2.at[slot],
                         sems.at[slot])

    def drain(slot):
        dummy = emb2_hbm.at[pl.ds(0, CG)]
        pltpu.make_async_copy(dummy, rows1.at[slot], sems.at[slot]).wait()
        pltpu.make_async_copy(dummy, rows2.at[slot], sems.at[slot]).wait()

    iota = lax.iota(jnp.int32, L)
    fire(0, 0)

    for j in range(NCHUNK):
        slot = j % 2
        drain(slot)
        if j + 1 < NCHUNK:
            fire(j + 1, (j + 1) % 2)

        def group_body(h, _, j=j, slot=slot):
            off = pl.ds(j * CG + h * L, L)
            iv = h * L + iota
            s1 = jnp.where(ix1[off] >= half, D, 0).astype(jnp.int32)
            s2 = jnp.where(ix2[off] >= half, D, 0).astype(jnp.int32)
            acc = jnp.zeros((L,), jnp.float32)
            for d in range(D):
                skew = jnp.bitwise_and(iota + d, D - 1)
                a = plsc.load_gather(rows1, [jnp.full((L,), slot, jnp.int32),
                                             iv, s1 + skew])
                b = plsc.load_gather(rows2, [jnp.full((L,), slot, jnp.int32),
                                             iv, s2 + skew])
                acc = acc + a * b
            dots[off] = acc
            return 0

        lax.fori_loop(0, GPC, group_body, 0)

    pltpu.sync_copy(dots, out_hbm.at[pl.ds(base, BPW)])


_TB = 512
_NROWS = 500736
_HOFF = 499712
_TGRID = _NROWS // _TB


def _relayout_kernel(lo_ref, hi_ref, o_ref):
    eye = (lax.broadcasted_iota(jnp.int32, (D, D), 0)
           == lax.broadcasted_iota(jnp.int32, (D, D), 1)).astype(jnp.float32)
    dn = (((0,), (0,)), ((), ()))
    o_ref[:, 0:D] = lax.dot_general(lo_ref[...], eye, dn,
                                    preferred_element_type=jnp.float32)
    o_ref[:, D:2 * D] = lax.dot_general(hi_ref[...], eye, dn,
                                        preferred_element_type=jnp.float32)


def _relayout(embT):
    return pl.pallas_call(
        _relayout_kernel,
        grid=(_TGRID,),
        in_specs=[
            pl.BlockSpec((D, _TB), lambda j: (0, j)),
            pl.BlockSpec((D, _TB), lambda j: (0, j + _HOFF // _TB)),
        ],
        out_specs=pl.BlockSpec((_TB, 128), lambda j: (j, 0)),
        out_shape=jax.ShapeDtypeStruct((_NROWS, 128), jnp.float32),
    )(embT, embT)


def _loss_kernel(d_ref, w_ref, o_ref):
    x = w_ref[...] * d_ref[...]
    y = jnp.minimum(x, 0.0) - jnp.log1p(jnp.exp(-jnp.abs(x)))
    o_ref[0, 0] = -jnp.sum(y) * (1.0 / B)


def _loss(dots, w):
    out = pl.pallas_call(
        _loss_kernel,
        out_shape=jax.ShapeDtypeStruct((1, 1), jnp.float32),
        out_specs=pl.BlockSpec(memory_space=pltpu.SMEM),
    )(dots.reshape(128, 128), w.reshape(128, 128))
    return out[0, 0]


def kernel(x1, x2, w, emb):
    emb2 = _relayout(emb.T)
    dots = _sc_dots(emb2, x1.astype(jnp.int32), x2.astype(jnp.int32))
    return _loss(dots, w)


# speedup vs baseline: 2.1085x; 2.1085x over previous
import functools

import jax
import jax.numpy as jnp
from jax import lax
from jax.experimental import pallas as pl
from jax.experimental.pallas import tpu as pltpu
from jax.experimental.pallas import tpu_sc as plsc

B = 16384
D = 64
NC, NS, L = 2, 16, 16
NW = NC * NS
BPW = B // NW
CG = 128
NCHUNK = BPW // CG
GPC = CG // L

_mesh = plsc.VectorSubcoreMesh(core_axis_name="c", subcore_axis_name="s")


@functools.partial(
    pl.kernel,
    out_type=jax.ShapeDtypeStruct((B,), jnp.float32),
    mesh=_mesh,
    scratch_types=[
        pltpu.VMEM((BPW,), jnp.int32),
        pltpu.VMEM((BPW,), jnp.int32),
        pltpu.VMEM((BPW,), jnp.int32),
        pltpu.VMEM((BPW,), jnp.int32),
        pltpu.VMEM((2, CG, 128), jnp.float32),
        pltpu.VMEM((2, CG, 128), jnp.float32),
        pltpu.VMEM((BPW,), jnp.float32),
        pltpu.SemaphoreType.DMA((2,)),
    ],
    compiler_params=pltpu.CompilerParams(needs_layout_passes=False),
)
def _sc_dots(emb2_hbm, x1_hbm, x2_hbm, out_hbm, ix1, ix2, g1, g2,
             rows1, rows2, dots, sems):
    wid = lax.axis_index("s") * NC + lax.axis_index("c")
    base = wid * BPW

    pltpu.sync_copy(x1_hbm.at[pl.ds(base, BPW)], ix1)
    pltpu.sync_copy(x2_hbm.at[pl.ds(base, BPW)], ix2)
    half = jnp.full((L,), 500000, jnp.int32)
    hoff = jnp.full((L,), 499712, jnp.int32)
    for k in range(BPW // L):
        sl = pl.ds(k * L, L)
        v1 = ix1[sl]
        v2 = ix2[sl]
        g1[sl] = jnp.where(v1 >= half, v1 - hoff, v1)
        g2[sl] = jnp.where(v2 >= half, v2 - hoff, v2)

    def fire(j, slot):
        isl = pl.ds(j * CG, CG)
        pltpu.async_copy(emb2_hbm.at[g1.at[isl]], rows1.at[slot],
                         sems.at[slot])
        pltpu.async_copy(emb2_hbm.at[g2.at[isl]], rows2.at[slot],
                         sems.at[slot])

    def drain(slot):
        dummy = emb2_hbm.at[pl.ds(0, CG)]
        pltpu.make_async_copy(dummy, rows1.at[slot], sems.at[slot]).wait()
        pltpu.make_async_copy(dummy, rows2.at[slot], sems.at[slot]).wait()

    iota = lax.iota(jnp.int32, L)
    fire(0, 0)

    for j in range(NCHUNK):
        slot = j % 2
        drain(slot)
        if j + 1 < NCHUNK:
            fire(j + 1, (j + 1) % 2)

        def group_body(h, _, j=j, slot=slot):
            off = pl.ds(j * CG + h * L, L)
            iv = h * L + iota
            s1 = jnp.where(ix1[off] >= half, D, 0).astype(jnp.int32)
            s2 = jnp.where(ix2[off] >= half, D, 0).astype(jnp.int32)
            acc = jnp.zeros((L,), jnp.float32)
            for d in range(D):
                skew = jnp.bitwise_and(iota + d, D - 1)
                a = plsc.load_gather(rows1, [jnp.full((L,), slot, jnp.int32),
                                             iv, s1 + skew])
                b = plsc.load_gather(rows2, [jnp.full((L,), slot, jnp.int32),
                                             iv, s2 + skew])
                acc = acc + a * b
            dots[off] = acc
            return 0

        lax.fori_loop(0, GPC, group_body, 0)

    pltpu.sync_copy(dots, out_hbm.at[pl.ds(base, BPW)])


_TB = 2048
_NROWS = 501760
_HOFF = 499712
_TGRID = _NROWS // _TB


def _relayout_kernel(lo_ref, hi_ref, o_ref):
    eye = (lax.broadcasted_iota(jnp.int32, (D, D), 0)
           == lax.broadcasted_iota(jnp.int32, (D, D), 1)).astype(jnp.float32)
    dn = (((0,), (0,)), ((), ()))
    o_ref[:, 0:D] = lax.dot_general(lo_ref[...], eye, dn,
                                    preferred_element_type=jnp.float32)
    o_ref[:, D:2 * D] = lax.dot_general(hi_ref[...], eye, dn,
                                        preferred_element_type=jnp.float32)


def _relayout(embT):
    return pl.pallas_call(
        _relayout_kernel,
        grid=(_TGRID,),
        in_specs=[
            pl.BlockSpec((D, _TB), lambda j: (0, j)),
            pl.BlockSpec((D, _TB), lambda j: (0, j + _HOFF // _TB)),
        ],
        out_specs=pl.BlockSpec((_TB, 128), lambda j: (j, 0)),
        out_shape=jax.ShapeDtypeStruct((_NROWS, 128), jnp.float32),
        compiler_params=pltpu.CompilerParams(
            fuse_transposed_lhs_in_matmul=True),
    )(embT, embT)


def _loss_kernel(d_ref, w_ref, o_ref):
    x = w_ref[...] * d_ref[...]
    y = jnp.minimum(x, 0.0) - jnp.log1p(jnp.exp(-jnp.abs(x)))
    o_ref[0, 0] = -jnp.sum(y) * (1.0 / B)


def _loss(dots, w):
    out = pl.pallas_call(
        _loss_kernel,
        out_shape=jax.ShapeDtypeStruct((1, 1), jnp.float32),
        out_specs=pl.BlockSpec(memory_space=pltpu.SMEM),
    )(dots.reshape(128, 128), w.reshape(128, 128))
    return out[0, 0]


def kernel(x1, x2, w, emb):
    emb2 = _relayout(emb.T)
    dots = _sc_dots(emb2, x1.astype(jnp.int32), x2.astype(jnp.int32))
    return _loss(dots, w)


# speedup vs baseline: 2.6148x; 1.2401x over previous
import functools

import jax
import jax.numpy as jnp
from jax import lax
from jax.experimental import pallas as pl
from jax.experimental.pallas import tpu as pltpu
from jax.experimental.pallas import tpu_sc as plsc

B = 16384
D = 64
NC, NS, L = 2, 16, 16
NW = NC * NS
BPW = B // NW
CG = 128
NCHUNK = BPW // CG
GPC = CG // L

_mesh = plsc.VectorSubcoreMesh(core_axis_name="c", subcore_axis_name="s")


@functools.partial(
    pl.kernel,
    out_type=jax.ShapeDtypeStruct((B,), jnp.float32),
    mesh=_mesh,
    scratch_types=[
        pltpu.VMEM((BPW,), jnp.int32),
        pltpu.VMEM((BPW,), jnp.int32),
        pltpu.VMEM((BPW,), jnp.int32),
        pltpu.VMEM((BPW,), jnp.int32),
        pltpu.VMEM((2, CG, 128), jnp.float32),
        pltpu.VMEM((2, CG, 128), jnp.float32),
        pltpu.VMEM((BPW,), jnp.float32),
        pltpu.SemaphoreType.DMA((2,)),
    ],
    compiler_params=pltpu.CompilerParams(needs_layout_passes=False),
)
def _sc_dots(emb2_hbm, x1_hbm, x2_hbm, out_hbm, ix1, ix2, g1, g2,
             rows1, rows2, dots, sems):
    wid = lax.axis_index("s") * NC + lax.axis_index("c")
    base = wid * BPW

    pltpu.sync_copy(x1_hbm.at[pl.ds(base, BPW)], ix1)
    pltpu.sync_copy(x2_hbm.at[pl.ds(base, BPW)], ix2)
    half = jnp.full((L,), 500000, jnp.int32)
    hoff = jnp.full((L,), 499712, jnp.int32)
    for k in range(BPW // L):
        sl = pl.ds(k * L, L)
        v1 = ix1[sl]
        v2 = ix2[sl]
        g1[sl] = jnp.where(v1 >= half, v1 - hoff, v1)
        g2[sl] = jnp.where(v2 >= half, v2 - hoff, v2)

    def fire(j, slot):
        isl = pl.ds(j * CG, CG)
        pltpu.async_copy(emb2_hbm.at[g1.at[isl]], rows1.at[slot],
                         sems.at[slot])
        pltpu.async_copy(emb2_hbm.at[g2.at[isl]], rows2.at[slot],
                         sems.at[slot])

    def drain(slot):
        dummy = emb2_hbm.at[pl.ds(0, CG)]
        pltpu.make_async_copy(dummy, rows1.at[slot], sems.at[slot]).wait()
        pltpu.make_async_copy(dummy, rows2.at[slot], sems.at[slot]).wait()

    iota = lax.iota(jnp.int32, L)
    fire(0, 0)

    for j in range(NCHUNK):
        slot = j % 2
        drain(slot)
        if j + 1 < NCHUNK:
            fire(j + 1, (j + 1) % 2)

        def group_body(h, _, j=j, slot=slot):
            off = pl.ds(j * CG + h * L, L)
            iv = h * L + iota
            s1 = jnp.where(ix1[off] >= half, D, 0).astype(jnp.int32)
            s2 = jnp.where(ix2[off] >= half, D, 0).astype(jnp.int32)
            acc = jnp.zeros((L,), jnp.float32)
            for d in range(D):
                skew = jnp.bitwise_and(iota + d, D - 1)
                a = plsc.load_gather(rows1, [jnp.full((L,), slot, jnp.int32),
                                             iv, s1 + skew])
                b = plsc.load_gather(rows2, [jnp.full((L,), slot, jnp.int32),
                                             iv, s2 + skew])
                acc = acc + a * b
            dots[off] = acc
            return 0

        lax.fori_loop(0, GPC, group_body, 0)

    pltpu.sync_copy(dots, out_hbm.at[pl.ds(base, BPW)])


_TB = 4096
_NROWS = 503808
_HOFF = 499712
_TGRID = _NROWS // _TB


def _relayout_kernel(lo_ref, hi_ref, o_ref):
    o_ref[:, 0:D] = jnp.transpose(lo_ref[...], (1, 0))
    o_ref[:, D:2 * D] = jnp.transpose(hi_ref[...], (1, 0))


def _relayout(embT):
    return pl.pallas_call(
        _relayout_kernel,
        grid=(_TGRID,),
        in_specs=[
            pl.BlockSpec((D, _TB), lambda j: (0, j)),
            pl.BlockSpec((D, _TB), lambda j: (0, j + _HOFF // _TB)),
        ],
        out_specs=pl.BlockSpec((_TB, 128), lambda j: (j, 0)),
        out_shape=jax.ShapeDtypeStruct((_NROWS, 128), jnp.float32),
        compiler_params=pltpu.CompilerParams(
            fuse_transposed_lhs_in_matmul=True),
    )(embT, embT)


def _loss_kernel(d_ref, w_ref, o_ref):
    x = w_ref[...] * d_ref[...]
    y = jnp.minimum(x, 0.0) - jnp.log1p(jnp.exp(-jnp.abs(x)))
    o_ref[0, 0] = -jnp.sum(y) * (1.0 / B)


def _loss(dots, w):
    out = pl.pallas_call(
        _loss_kernel,
        out_shape=jax.ShapeDtypeStruct((1, 1), jnp.float32),
        out_specs=pl.BlockSpec(memory_space=pltpu.SMEM),
    )(dots.reshape(128, 128), w.reshape(128, 128))
    return out[0, 0]


def kernel(x1, x2, w, emb):
    emb2 = _relayout(emb.T)
    dots = _sc_dots(emb2, x1.astype(jnp.int32), x2.astype(jnp.int32))
    return _loss(dots, w)


# speedup vs baseline: 4.0152x; 1.5355x over previous
import functools

import jax
import jax.numpy as jnp
from jax import lax
from jax.experimental import pallas as pl
from jax.experimental.pallas import tpu as pltpu
from jax.experimental.pallas import tpu_sc as plsc

B = 16384
D = 64
NC, NS, L = 2, 16, 16
NW = NC * NS
BPW = B // NW
CG = 128
NCHUNK = BPW // CG
GPC = CG // L

_mesh = plsc.VectorSubcoreMesh(core_axis_name="c", subcore_axis_name="s")


@functools.partial(
    pl.kernel,
    out_type=jax.ShapeDtypeStruct((B,), jnp.float32),
    mesh=_mesh,
    scratch_types=[
        pltpu.VMEM((BPW,), jnp.int32),
        pltpu.VMEM((BPW,), jnp.int32),
        pltpu.VMEM((BPW,), jnp.int32),
        pltpu.VMEM((BPW,), jnp.int32),
        pltpu.VMEM((2, CG, 128), jnp.int32),
        pltpu.VMEM((2, CG, 128), jnp.int32),
        pltpu.VMEM((BPW,), jnp.float32),
        pltpu.SemaphoreType.DMA((2,)),
    ],
    compiler_params=pltpu.CompilerParams(needs_layout_passes=False),
)
def _sc_dots(emb2_hbm, x1_hbm, x2_hbm, out_hbm, ix1, ix2, g1, g2,
             rows1, rows2, dots, sems):
    wid = lax.axis_index("s") * NC + lax.axis_index("c")
    base = wid * BPW

    pltpu.sync_copy(x1_hbm.at[pl.ds(base, BPW)], ix1)
    pltpu.sync_copy(x2_hbm.at[pl.ds(base, BPW)], ix2)
    def quarter_base(v):
        b = jnp.where(v >= 250000, 249856, 0)
        b = jnp.where(v >= 500000, 499712, b)
        return jnp.where(v >= 750000, 749568, b)

    def quarter_col(v):
        c = jnp.where(v >= 250000, 32, 0)
        c = jnp.where(v >= 500000, 64, c)
        return jnp.where(v >= 750000, 96, c)

    for k in range(BPW // L):
        sl = pl.ds(k * L, L)
        g1[sl] = ix1[sl] - quarter_base(ix1[sl])
        g2[sl] = ix2[sl] - quarter_base(ix2[sl])

    def fire(j, slot):
        isl = pl.ds(j * CG, CG)
        pltpu.async_copy(emb2_hbm.at[g1.at[isl]], rows1.at[slot],
                         sems.at[slot])
        pltpu.async_copy(emb2_hbm.at[g2.at[isl]], rows2.at[slot],
                         sems.at[slot])

    def drain(slot):
        dummy = emb2_hbm.at[pl.ds(0, CG)]
        pltpu.make_async_copy(dummy, rows1.at[slot], sems.at[slot]).wait()
        pltpu.make_async_copy(dummy, rows2.at[slot], sems.at[slot]).wait()

    iota = lax.iota(jnp.int32, L)
    fire(0, 0)

    for j in range(NCHUNK):
        slot = j % 2
        drain(slot)
        if j + 1 < NCHUNK:
            fire(j + 1, (j + 1) % 2)

        def group_body(h, _, j=j, slot=slot):
            off = pl.ds(j * CG + h * L, L)
            iv = h * L + iota
            s1 = quarter_col(ix1[off])
            s2 = quarter_col(ix2[off])
            acc = jnp.zeros((L,), jnp.float32)
            himask = jnp.full((L,), -65536, jnp.int32)
            slotv = jnp.full((L,), slot, jnp.int32)
            for d in range(D // 2):
                skew = jnp.bitwise_and(iota + d, D // 2 - 1)
                a = plsc.load_gather(rows1, [slotv, iv, s1 + skew])
                b = plsc.load_gather(rows2, [slotv, iv, s2 + skew])
                alo = plsc.bitcast(jnp.left_shift(a, 16), jnp.float32)
                blo = plsc.bitcast(jnp.left_shift(b, 16), jnp.float32)
                ahi = plsc.bitcast(jnp.bitwise_and(a, himask), jnp.float32)
                bhi = plsc.bitcast(jnp.bitwise_and(b, himask), jnp.float32)
                acc = acc + alo * blo + ahi * bhi
            dots[off] = acc
            return 0

        lax.fori_loop(0, GPC, group_body, 0)

    pltpu.sync_copy(dots, out_hbm.at[pl.ds(base, BPW)])


_TB = 2048
_NROWS = 251904
_Q = (0, 249856, 499712, 749568)
_TGRID = _NROWS // _TB


def _relayout_kernel(q0_ref, q1_ref, q2_ref, q3_ref, o_ref):
    rnd = jnp.uint32(0x8000)
    himask = jnp.uint32(0xFFFF0000)
    cols = []
    for ref in (q0_ref, q1_ref, q2_ref, q3_ref):
        r = lax.bitcast_convert_type(ref[...], jnp.uint32) + rnd
        lo = jnp.right_shift(r[0:D // 2, :], 16)
        hi = jnp.bitwise_and(r[D // 2:D, :], himask)
        cols.append(jnp.bitwise_or(lo, hi))
    packed = jnp.concatenate(cols, axis=0)
    tr = jnp.transpose(packed, (1, 0))
    o_ref[...] = lax.bitcast_convert_type(tr, jnp.int32)


def _relayout(embT):
    return pl.pallas_call(
        _relayout_kernel,
        grid=(_TGRID,),
        in_specs=[pl.BlockSpec((D, _TB), lambda j, q=q: (0, j + _Q[q] // _TB))
                  for q in range(4)],
        out_specs=pl.BlockSpec((_TB, 128), lambda j: (j, 0)),
        out_shape=jax.ShapeDtypeStruct((_NROWS, 128), jnp.int32),
    )(embT, embT, embT, embT)


def _loss_kernel(d_ref, w_ref, o_ref):
    x = w_ref[...] * d_ref[...]
    y = jnp.minimum(x, 0.0) - jnp.log1p(jnp.exp(-jnp.abs(x)))
    o_ref[0, 0] = -jnp.sum(y) * (1.0 / B)


def _loss(dots, w):
    out = pl.pallas_call(
        _loss_kernel,
        out_shape=jax.ShapeDtypeStruct((1, 1), jnp.float32),
        out_specs=pl.BlockSpec(memory_space=pltpu.SMEM),
    )(dots.reshape(128, 128), w.reshape(128, 128))
    return out[0, 0]


def kernel(x1, x2, w, emb):
    emb2 = _relayout(emb.T)
    dots = _sc_dots(emb2, x1.astype(jnp.int32), x2.astype(jnp.int32))
    return _loss(dots, w)


# speedup vs baseline: 4.7866x; 1.1921x over previous
import functools

import jax
import jax.numpy as jnp
from jax import lax
from jax.experimental import pallas as pl
from jax.experimental.pallas import tpu as pltpu
from jax.experimental.pallas import tpu_sc as plsc

B = 16384
D = 64
NC, NS, L = 2, 16, 16
NW = NC * NS
BPW = B // NW
CG = 128
NCHUNK = BPW // CG
GPC = CG // L

_mesh = plsc.VectorSubcoreMesh(core_axis_name="c", subcore_axis_name="s")


@functools.partial(
    pl.kernel,
    out_type=jax.ShapeDtypeStruct((B,), jnp.float32),
    mesh=_mesh,
    scratch_types=[
        pltpu.VMEM((BPW,), jnp.int32),
        pltpu.VMEM((BPW,), jnp.int32),
        pltpu.VMEM((BPW,), jnp.int32),
        pltpu.VMEM((BPW,), jnp.int32),
        pltpu.VMEM((2, CG, 128), jnp.int32),
        pltpu.VMEM((2, CG, 128), jnp.int32),
        pltpu.VMEM((BPW,), jnp.float32),
        pltpu.SemaphoreType.DMA((2,)),
    ],
    compiler_params=pltpu.CompilerParams(needs_layout_passes=False),
)
def _sc_dots(emb2_hbm, x1_hbm, x2_hbm, out_hbm, ix1, ix2, g1, g2,
             rows1, rows2, dots, sems):
    wid = lax.axis_index("s") * NC + lax.axis_index("c")
    base = wid * BPW

    pltpu.sync_copy(x1_hbm.at[pl.ds(base, BPW)], ix1)
    pltpu.sync_copy(x2_hbm.at[pl.ds(base, BPW)], ix2)
    def quarter_base(v):
        b = jnp.where(v >= 250000, 249856, 0)
        b = jnp.where(v >= 500000, 499712, b)
        return jnp.where(v >= 750000, 749568, b)

    def quarter_col(v):
        c = jnp.where(v >= 250000, 32, 0)
        c = jnp.where(v >= 500000, 64, c)
        return jnp.where(v >= 750000, 96, c)

    for k in range(BPW // L):
        sl = pl.ds(k * L, L)
        g1[sl] = ix1[sl] - quarter_base(ix1[sl])
        g2[sl] = ix2[sl] - quarter_base(ix2[sl])

    def fire(j, slot):
        isl = pl.ds(j * CG, CG)
        pltpu.async_copy(emb2_hbm.at[g1.at[isl]], rows1.at[slot],
                         sems.at[slot])
        pltpu.async_copy(emb2_hbm.at[g2.at[isl]], rows2.at[slot],
                         sems.at[slot])

    def drain(slot):
        dummy = emb2_hbm.at[pl.ds(0, CG)]
        pltpu.make_async_copy(dummy, rows1.at[slot], sems.at[slot]).wait()
        pltpu.make_async_copy(dummy, rows2.at[slot], sems.at[slot]).wait()

    iota = lax.iota(jnp.int32, L)
    fire(0, 0)

    for j in range(NCHUNK):
        slot = j % 2
        drain(slot)
        if j + 1 < NCHUNK:
            fire(j + 1, (j + 1) % 2)

        def group_body(h, _, j=j, slot=slot):
            off = pl.ds(j * CG + h * L, L)
            iv = h * L + iota
            s1 = quarter_col(ix1[off])
            s2 = quarter_col(ix2[off])
            acc = jnp.zeros((L,), jnp.float32)
            himask = jnp.full((L,), -65536, jnp.int32)
            slotv = jnp.full((L,), slot, jnp.int32)
            for d in range(D // 2):
                skew = jnp.bitwise_and(iota + d, D // 2 - 1)
                a = plsc.load_gather(rows1, [slotv, iv, s1 + skew])
                b = plsc.load_gather(rows2, [slotv, iv, s2 + skew])
                alo = plsc.bitcast(jnp.left_shift(a, 16), jnp.float32)
                blo = plsc.bitcast(jnp.left_shift(b, 16), jnp.float32)
                ahi = plsc.bitcast(jnp.bitwise_and(a, himask), jnp.float32)
                bhi = plsc.bitcast(jnp.bitwise_and(b, himask), jnp.float32)
                acc = acc + alo * blo + ahi * bhi
            dots[off] = acc
            return 0

        lax.fori_loop(0, GPC, group_body, 0)

    pltpu.sync_copy(dots, out_hbm.at[pl.ds(base, BPW)])


_TB = 4096
_NROWS = 253952
_Q = (0, 249856, 499712, 749568)
_TGRID = _NROWS // _TB


def _relayout_kernel(q0_ref, q1_ref, q2_ref, q3_ref, o_ref):
    rnd = jnp.uint32(0x8000)
    himask = jnp.uint32(0xFFFF0000)
    cols = []
    for ref in (q0_ref, q1_ref, q2_ref, q3_ref):
        r = lax.bitcast_convert_type(ref[...], jnp.uint32) + rnd
        lo = jnp.right_shift(r[0:D // 2, :], 16)
        hi = jnp.bitwise_and(r[D // 2:D, :], himask)
        cols.append(jnp.bitwise_or(lo, hi))
    packed = jnp.concatenate(cols, axis=0)
    tr = jnp.transpose(packed, (1, 0))
    o_ref[...] = lax.bitcast_convert_type(tr, jnp.int32)


def _relayout(embT):
    return pl.pallas_call(
        _relayout_kernel,
        grid=(_TGRID,),
        in_specs=[pl.BlockSpec((D, _TB), lambda j, q=q: (0, j + _Q[q] // _TB))
                  for q in range(4)],
        out_specs=pl.BlockSpec((_TB, 128), lambda j: (j, 0)),
        out_shape=jax.ShapeDtypeStruct((_NROWS, 128), jnp.int32),
    )(embT, embT, embT, embT)


def _loss_kernel(d_ref, w_ref, o_ref):
    x = w_ref[...] * d_ref[...]
    y = jnp.minimum(x, 0.0) - jnp.log1p(jnp.exp(-jnp.abs(x)))
    o_ref[0, 0] = -jnp.sum(y) * (1.0 / B)


def _loss(dots, w):
    out = pl.pallas_call(
        _loss_kernel,
        out_shape=jax.ShapeDtypeStruct((1, 1), jnp.float32),
        out_specs=pl.BlockSpec(memory_space=pltpu.SMEM),
    )(dots.reshape(128, 128), w.reshape(128, 128))
    return out[0, 0]


def kernel(x1, x2, w, emb):
    emb2 = _relayout(emb.T)
    dots = _sc_dots(emb2, x1.astype(jnp.int32), x2.astype(jnp.int32))
    return _loss(dots, w)
